# Initial kernel scaffold; baseline (speedup 1.0000x reference)
import jax
import jax.numpy as jnp
from jax.experimental import pallas as pl


def kernel(x, edge_index, edge_attr, batch, Wr1, br1, Wr2, br2, Wk, W3, b3, W4, b4):
    raise NotImplementedError("write your pallas kernel here")


# speedup vs baseline: 3.4616x; 3.4616x over previous
import functools

import jax
import jax.numpy as jnp
from jax import lax
from jax.experimental import pallas as pl
from jax.experimental.pallas import tpu as pltpu
from jax.experimental.pallas import tpu_sc as plsc

N = 10000
E = 320000
HDIM = 128
EDIM = 16
ZDIM = 32
NODE_NUM = 100
LAYERS = 3

NUM_SC = 2
NUM_TILES = 16
NW = NUM_SC * NUM_TILES
EPW = E // NW
K = 80
NCHUNK = EPW // K
ROWS_PER_TILE = 640
PADN = ROWS_PER_TILE * NUM_TILES

_F32 = jnp.float32


def _dense0_body(h_ref, wki_ref, wkj_ref, wr1_ref, br1_ref, wr2_ref, br2_ref,
                 a_ref, b_ref, r_ref):
    h = h_ref[...]
    a_ref[...] = jnp.dot(h, wki_ref[...], preferred_element_type=_F32)
    b_ref[...] = jnp.dot(h, wkj_ref[...], preferred_element_type=_F32)
    t = jnp.dot(h, wr1_ref[...], preferred_element_type=_F32) + br1_ref[...]
    r_ref[...] = jnp.dot(t, wr2_ref[...], preferred_element_type=_F32) + br2_ref[...]


def _denseL_body(rp_ref, agg_ref, wki_ref, wkj_ref, wr1_ref, br1_ref, wr2_ref,
                 br2_ref, a_ref, b_ref, r_ref):
    h = rp_ref[...] + agg_ref[0] + agg_ref[1]
    a_ref[...] = jnp.dot(h, wki_ref[...], preferred_element_type=_F32)
    b_ref[...] = jnp.dot(h, wkj_ref[...], preferred_element_type=_F32)
    t = jnp.dot(h, wr1_ref[...], preferred_element_type=_F32) + br1_ref[...]
    r_ref[...] = jnp.dot(t, wr2_ref[...], preferred_element_type=_F32) + br2_ref[...]


_BR = 1000

_W_SPEC = pl.BlockSpec((HDIM, HDIM), lambda i: (0, 0))
_BIAS_SPEC = pl.BlockSpec((1, HDIM), lambda i: (0, 0))
_ROW_SPEC = pl.BlockSpec((_BR, HDIM), lambda i: (i, 0))
_AGG_SPEC = pl.BlockSpec((NUM_SC, _BR, HDIM), lambda i: (0, i, 0))
_OUT3 = [jax.ShapeDtypeStruct((N, HDIM), _F32)] * 3


def _dense0(h, wki, wkj, wr1, br1, wr2, br2):
    return pl.pallas_call(
        _dense0_body,
        grid=(N // _BR,),
        in_specs=[_ROW_SPEC, _W_SPEC, _W_SPEC, _W_SPEC, _BIAS_SPEC, _W_SPEC,
                  _BIAS_SPEC],
        out_specs=[_ROW_SPEC, _ROW_SPEC, _ROW_SPEC],
        out_shape=_OUT3,
    )(h, wki, wkj, wr1, br1, wr2, br2)


def _denseL(r_prev, agg, wki, wkj, wr1, br1, wr2, br2):
    return pl.pallas_call(
        _denseL_body,
        grid=(N // _BR,),
        in_specs=[_ROW_SPEC, _AGG_SPEC, _W_SPEC, _W_SPEC, _W_SPEC, _BIAS_SPEC,
                  _W_SPEC, _BIAS_SPEC],
        out_specs=[_ROW_SPEC, _ROW_SPEC, _ROW_SPEC],
        out_shape=_OUT3,
    )(r_prev, agg, wki, wkj, wr1, br1, wr2, br2)


def _edgec_body(ea_ref, wke_ref, c_ref):
    c_ref[...] = jnp.dot(ea_ref[...], wke_ref[...], preferred_element_type=_F32)


_BE = 2000


def _edge_c(ea, wke):
    return pl.pallas_call(
        _edgec_body,
        grid=(E // _BE,),
        in_specs=[pl.BlockSpec((_BE, EDIM), lambda i: (i, 0)),
                  pl.BlockSpec((EDIM, HDIM), lambda i: (0, 0))],
        out_specs=pl.BlockSpec((_BE, HDIM), lambda i: (i, 0)),
        out_shape=jax.ShapeDtypeStruct((E, HDIM), _F32),
    )(ea, wke)


def _pool_body(rp_ref, agg_ref, w3_ref, b3_ref, w4_ref, b4_ref, mu_ref, lv_ref):
    h = rp_ref[...] + agg_ref[0] + agg_ref[1]
    hg = jnp.sum(h, axis=1)
    mu_ref[...] = jnp.dot(hg, w3_ref[...], preferred_element_type=_F32) + b3_ref[...]
    lv_ref[...] = jnp.dot(hg, w4_ref[...], preferred_element_type=_F32) + b4_ref[...]


def _pool(r_prev, agg, w3, b3, w4, b4):
    ngraph = N // NODE_NUM
    return pl.pallas_call(
        _pool_body,
        out_shape=[jax.ShapeDtypeStruct((ngraph, ZDIM), _F32)] * 2,
    )(r_prev.reshape(ngraph, NODE_NUM, HDIM),
      agg.reshape(NUM_SC, ngraph, NODE_NUM, HDIM), w3, b3, w4, b4)


_ZROWS = 128


def _edge_body(a_hbm, b_hbm, c_hbm, dst_hbm, src_hbm, out_hbm,
               dstv, srcv, arow, brow, crow, zrow, aggsh, sema, semb):
    c = lax.axis_index("c")
    s = lax.axis_index("s")
    wid = c * NUM_TILES + s

    def zfill(i, carry):
        for j in range(HDIM // 16):
            zrow[i, pl.ds(j * 16, 16)] = jnp.zeros((16,), _F32)
        return carry
    lax.fori_loop(0, _ZROWS, zfill, 0)

    def zcopy(i, carry):
        pltpu.sync_copy(zrow, aggsh.at[pl.ds(s * ROWS_PER_TILE + i * _ZROWS, _ZROWS)])
        return carry
    lax.fori_loop(0, ROWS_PER_TILE // _ZROWS, zcopy, 0)
    plsc.subcore_barrier()

    ebase = wid * EPW

    def chunk(g, carry):
        base = ebase + g * K
        pltpu.sync_copy(dst_hbm.at[pl.ds(base, K)], dstv)
        pltpu.sync_copy(src_hbm.at[pl.ds(base, K)], srcv)
        cp_a = pltpu.async_copy(a_hbm.at[dstv], arow, sema)
        cp_b = pltpu.async_copy(b_hbm.at[srcv], brow, semb)
        pltpu.sync_copy(c_hbm.at[pl.ds(base, K)], crow)
        cp_a.wait()
        cp_b.wait()

        def edge(e, ecarry):
            for j in range(HDIM // 16):
                sl = pl.ds(j * 16, 16)
                t = arow[e, sl] + brow[e, sl] + crow[e, sl]
                arow[e, sl] = jnp.where(t >= 0.0, t, t * _F32(0.01))
            return ecarry
        lax.fori_loop(0, K, edge, 0)

        pltpu.sync_copy(arow, aggsh.at[dstv], add=True)
        return carry
    lax.fori_loop(0, NCHUNK, chunk, 0)

    plsc.subcore_barrier()
    pltpu.sync_copy(aggsh.at[pl.ds(s * ROWS_PER_TILE, ROWS_PER_TILE)],
                    out_hbm.at[c, pl.ds(s * ROWS_PER_TILE, ROWS_PER_TILE)])


_edge_kernel = functools.partial(
    pl.kernel,
    out_type=jax.ShapeDtypeStruct((NUM_SC, PADN, HDIM), _F32),
    mesh=plsc.VectorSubcoreMesh(core_axis_name="c", subcore_axis_name="s",
                                num_cores=NUM_SC, num_subcores=NUM_TILES),
    scratch_types=[
        pltpu.VMEM((K,), jnp.int32),
        pltpu.VMEM((K,), jnp.int32),
        pltpu.VMEM((K, HDIM), _F32),
        pltpu.VMEM((K, HDIM), _F32),
        pltpu.VMEM((K, HDIM), _F32),
        pltpu.VMEM((_ZROWS, HDIM), _F32),
        pltpu.VMEM_SHARED((PADN, HDIM), _F32),
        pltpu.SemaphoreType.DMA,
        pltpu.SemaphoreType.DMA,
    ],
)(_edge_body)


def kernel(x, edge_index, edge_attr, batch, Wr1, br1, Wr2, br2, Wk, W3, b3,
           W4, b4):
    del batch
    src = edge_index[0].astype(jnp.int32)
    dst = edge_index[1].astype(jnp.int32)

    r_prev = None
    agg = None
    for l in range(LAYERS):
        wki = Wk[l, :HDIM, :]
        wkj = Wk[l, HDIM:2 * HDIM, :]
        wke = Wk[l, 2 * HDIM:, :]
        br1l = br1[l].reshape(1, HDIM)
        br2l = br2[l].reshape(1, HDIM)
        if l == 0:
            a, b, r = _dense0(x, wki, wkj, Wr1[l], br1l, Wr2[l], br2l)
        else:
            a, b, r = _denseL(r_prev, agg, wki, wkj, Wr1[l], br1l, Wr2[l], br2l)
        cmat = _edge_c(edge_attr, wke)
        agg = _edge_kernel(a, b, cmat, dst, src)
        r_prev = r

    mu, logvar = _pool(r_prev, agg[:, :N, :], W3, b3.reshape(1, ZDIM), W4,
                       b4.reshape(1, ZDIM))
    return (mu, logvar)


# speedup vs baseline: 3.5093x; 1.0138x over previous
import functools

import jax
import jax.numpy as jnp
from jax import lax
from jax.experimental import pallas as pl
from jax.experimental.pallas import tpu as pltpu
from jax.experimental.pallas import tpu_sc as plsc

N = 10000
E = 320000
HDIM = 128
EDIM = 16
ZDIM = 32
NODE_NUM = 100
LAYERS = 3

NUM_SC = 2
NUM_TILES = 16
NW = NUM_SC * NUM_TILES
EPW = E // NW
K = 40
NCHUNK = EPW // K
ROWS_PER_TILE = 640
PADN = ROWS_PER_TILE * NUM_TILES

_F32 = jnp.float32


def _dense0_body(h_ref, wki_ref, wkj_ref, wr1_ref, br1_ref, wr2_ref, br2_ref,
                 a_ref, b_ref, r_ref):
    h = h_ref[...]
    a_ref[...] = jnp.dot(h, wki_ref[...], preferred_element_type=_F32)
    b_ref[...] = jnp.dot(h, wkj_ref[...], preferred_element_type=_F32)
    t = jnp.dot(h, wr1_ref[...], preferred_element_type=_F32) + br1_ref[...]
    r_ref[...] = jnp.dot(t, wr2_ref[...], preferred_element_type=_F32) + br2_ref[...]


def _denseL_body(rp_ref, agg_ref, wki_ref, wkj_ref, wr1_ref, br1_ref, wr2_ref,
                 br2_ref, a_ref, b_ref, r_ref):
    h = rp_ref[...] + agg_ref[0] + agg_ref[1]
    a_ref[...] = jnp.dot(h, wki_ref[...], preferred_element_type=_F32)
    b_ref[...] = jnp.dot(h, wkj_ref[...], preferred_element_type=_F32)
    t = jnp.dot(h, wr1_ref[...], preferred_element_type=_F32) + br1_ref[...]
    r_ref[...] = jnp.dot(t, wr2_ref[...], preferred_element_type=_F32) + br2_ref[...]


_BR = 1000

_W_SPEC = pl.BlockSpec((HDIM, HDIM), lambda i: (0, 0))
_BIAS_SPEC = pl.BlockSpec((1, HDIM), lambda i: (0, 0))
_ROW_SPEC = pl.BlockSpec((_BR, HDIM), lambda i: (i, 0))
_AGG_SPEC = pl.BlockSpec((NUM_SC, _BR, HDIM), lambda i: (0, i, 0))
_OUT3 = [jax.ShapeDtypeStruct((N, HDIM), _F32)] * 3


def _dense0(h, wki, wkj, wr1, br1, wr2, br2):
    return pl.pallas_call(
        _dense0_body,
        grid=(N // _BR,),
        in_specs=[_ROW_SPEC, _W_SPEC, _W_SPEC, _W_SPEC, _BIAS_SPEC, _W_SPEC,
                  _BIAS_SPEC],
        out_specs=[_ROW_SPEC, _ROW_SPEC, _ROW_SPEC],
        out_shape=_OUT3,
    )(h, wki, wkj, wr1, br1, wr2, br2)


def _denseL(r_prev, agg, wki, wkj, wr1, br1, wr2, br2):
    return pl.pallas_call(
        _denseL_body,
        grid=(N // _BR,),
        in_specs=[_ROW_SPEC, _AGG_SPEC, _W_SPEC, _W_SPEC, _W_SPEC, _BIAS_SPEC,
                  _W_SPEC, _BIAS_SPEC],
        out_specs=[_ROW_SPEC, _ROW_SPEC, _ROW_SPEC],
        out_shape=_OUT3,
    )(r_prev, agg, wki, wkj, wr1, br1, wr2, br2)


def _edgec_body(ea_ref, wke_ref, c_ref):
    c_ref[...] = jnp.dot(ea_ref[...], wke_ref[...], preferred_element_type=_F32)


_BE = 2000


def _edge_c(ea, wke):
    return pl.pallas_call(
        _edgec_body,
        grid=(E // _BE,),
        in_specs=[pl.BlockSpec((_BE, EDIM), lambda i: (i, 0)),
                  pl.BlockSpec((EDIM, HDIM), lambda i: (0, 0))],
        out_specs=pl.BlockSpec((_BE, HDIM), lambda i: (i, 0)),
        out_shape=jax.ShapeDtypeStruct((E, HDIM), _F32),
    )(ea, wke)


def _pool_body(rp_ref, agg_ref, w3_ref, b3_ref, w4_ref, b4_ref, mu_ref, lv_ref):
    h = rp_ref[...] + agg_ref[0] + agg_ref[1]
    hg = jnp.sum(h, axis=1)
    mu_ref[...] = jnp.dot(hg, w3_ref[...], preferred_element_type=_F32) + b3_ref[...]
    lv_ref[...] = jnp.dot(hg, w4_ref[...], preferred_element_type=_F32) + b4_ref[...]


def _pool(r_prev, agg, w3, b3, w4, b4):
    ngraph = N // NODE_NUM
    return pl.pallas_call(
        _pool_body,
        out_shape=[jax.ShapeDtypeStruct((ngraph, ZDIM), _F32)] * 2,
    )(r_prev.reshape(ngraph, NODE_NUM, HDIM),
      agg.reshape(NUM_SC, ngraph, NODE_NUM, HDIM), w3, b3, w4, b4)


_ZROWS = 64


def _edge_body(a_hbm, b_hbm, c_hbm, dst_hbm, src_hbm, out_hbm,
               dstv, srcv, arow, brow, crow, zrow, aggsh,
               sema, semb, semc, semsc):
    c = lax.axis_index("c")
    s = lax.axis_index("s")
    wid = c * NUM_TILES + s

    def zfill(i, carry):
        for j in range(HDIM // 16):
            zrow[i, pl.ds(j * 16, 16)] = jnp.zeros((16,), _F32)
        return carry
    lax.fori_loop(0, _ZROWS, zfill, 0)

    def zcopy(i, carry):
        pltpu.sync_copy(zrow, aggsh.at[pl.ds(s * ROWS_PER_TILE + i * _ZROWS, _ZROWS)])
        return carry
    lax.fori_loop(0, ROWS_PER_TILE // _ZROWS, zcopy, 0)
    plsc.subcore_barrier()

    ebase = wid * EPW

    def fetch(g, p):
        base = ebase + g * K
        pltpu.sync_copy(dst_hbm.at[pl.ds(base, K)], dstv.at[p])
        pltpu.sync_copy(src_hbm.at[pl.ds(base, K)], srcv.at[p])
        pltpu.async_copy(a_hbm.at[dstv.at[p]], arow.at[p], sema)
        pltpu.async_copy(b_hbm.at[srcv.at[p]], brow.at[p], semb)
        pltpu.async_copy(c_hbm.at[pl.ds(base, K)], crow.at[p], semc)

    def wait_fetch(p):
        pltpu.make_async_copy(a_hbm.at[dstv.at[p]], arow.at[p], sema).wait()
        pltpu.make_async_copy(b_hbm.at[srcv.at[p]], brow.at[p], semb).wait()
        pltpu.make_async_copy(c_hbm.at[pl.ds(0, K)], crow.at[p], semc).wait()

    fetch(0, 0)

    def chunk2(g2, carry):
        for b in range(2):
            g = g2 * 2 + b
            wait_fetch(b)
            gnext = lax.min(g + 1, NCHUNK - 1)
            fetch(gnext, 1 - b)

            def edge(e, ecarry, _b=b):
                for j in range(HDIM // 16):
                    sl = pl.ds(j * 16, 16)
                    t = arow[_b, e, sl] + brow[_b, e, sl] + crow[_b, e, sl]
                    arow[_b, e, sl] = jnp.where(t >= 0.0, t, t * _F32(0.01))
                return ecarry
            lax.fori_loop(0, K, edge, 0)

            pltpu.async_copy(arow.at[b], aggsh.at[dstv.at[b]], semsc,
                             add=True).wait()
        return carry
    lax.fori_loop(0, NCHUNK // 2, chunk2, 0)
    wait_fetch(0)

    plsc.subcore_barrier()
    pltpu.sync_copy(aggsh.at[pl.ds(s * ROWS_PER_TILE, ROWS_PER_TILE)],
                    out_hbm.at[c, pl.ds(s * ROWS_PER_TILE, ROWS_PER_TILE)])


_edge_kernel = functools.partial(
    pl.kernel,
    out_type=jax.ShapeDtypeStruct((NUM_SC, PADN, HDIM), _F32),
    mesh=plsc.VectorSubcoreMesh(core_axis_name="c", subcore_axis_name="s",
                                num_cores=NUM_SC, num_subcores=NUM_TILES),
    scratch_types=[
        pltpu.VMEM((2, K), jnp.int32),
        pltpu.VMEM((2, K), jnp.int32),
        pltpu.VMEM((2, K, HDIM), _F32),
        pltpu.VMEM((2, K, HDIM), _F32),
        pltpu.VMEM((2, K, HDIM), _F32),
        pltpu.VMEM((_ZROWS, HDIM), _F32),
        pltpu.VMEM_SHARED((PADN, HDIM), _F32),
        pltpu.SemaphoreType.DMA,
        pltpu.SemaphoreType.DMA,
        pltpu.SemaphoreType.DMA,
        pltpu.SemaphoreType.DMA,
    ],
)(_edge_body)


def kernel(x, edge_index, edge_attr, batch, Wr1, br1, Wr2, br2, Wk, W3, b3,
           W4, b4):
    del batch
    src = edge_index[0].astype(jnp.int32)
    dst = edge_index[1].astype(jnp.int32)

    r_prev = None
    agg = None
    for l in range(LAYERS):
        wki = Wk[l, :HDIM, :]
        wkj = Wk[l, HDIM:2 * HDIM, :]
        wke = Wk[l, 2 * HDIM:, :]
        br1l = br1[l].reshape(1, HDIM)
        br2l = br2[l].reshape(1, HDIM)
        if l == 0:
            a, b, r = _dense0(x, wki, wkj, Wr1[l], br1l, Wr2[l], br2l)
        else:
            a, b, r = _denseL(r_prev, agg, wki, wkj, Wr1[l], br1l, Wr2[l], br2l)
        cmat = _edge_c(edge_attr, wke)
        agg = _edge_kernel(a, b, cmat, dst, src)
        r_prev = r

    mu, logvar = _pool(r_prev, agg[:, :N, :], W3, b3.reshape(1, ZDIM), W4,
                       b4.reshape(1, ZDIM))
    return (mu, logvar)
